# Initial kernel scaffold; baseline (speedup 1.0000x reference)
import jax
import jax.numpy as jnp
from jax.experimental import pallas as pl


def kernel(x, r_dw, r_gn1_g, r_gn1_b, r_pw1, r_gn2_g, r_gn2_b, r_pw2_w, r_pw2_b, e_w1, e_gn1_g, e_gn1_b, e_w2, e_gn2_g, e_gn2_b):
    raise NotImplementedError("write your pallas kernel here")


# speedup vs baseline: 12.0617x; 12.0617x over previous
import numpy as np

import jax
import jax.numpy as jnp
from jax.experimental import pallas as pl


def _np_constants(C, H, W, PS, RED, HID, E):
    HP, WP = H // PS, W // PS
    S = HP * WP
    l = np.arange(H * W)
    hh, ww = l // W, l % W
    PM = (((hh[:, None] // PS) * WP + (ww[:, None] // PS))
          == np.arange(S)[None, :]).astype(np.float32) / (PS * PS)
    si, sj = np.arange(S)[:, None] // WP, np.arange(S)[:, None] % WP
    oi, oj = np.arange(S)[None, :] // WP, np.arange(S)[None, :] % WP
    TT = np.zeros((9 * S, S), np.float32)
    for ky in range(3):
        for kx in range(3):
            t = ky * 3 + kx
            TT[t * S:(t + 1) * S] = ((si == oi + ky - 1) &
                                     (sj == oj + kx - 1)).astype(np.float32)

    def gn_ops(nch, ngrp, nspatial):
        g = (np.arange(nch)[None, :] // (nch // ngrp)
             == np.arange(ngrp)[:, None]).astype(np.float32)
        return g / (nch // ngrp * nspatial), g.T.copy()

    G1, U1 = gn_ops(C, 8, S)
    G2, U2 = gn_ops(RED, 3, S)
    GE1, UE1 = gn_ops(HID, 8, H * W)
    GE2, UE2 = gn_ops(C, 8, H * W)
    return PM, TT, G1, U1, G2, U2, GE1, UE1, GE2, UE2


def _gn(h, G, U, gamma, beta, eps=1e-5):
    m = jnp.sum(jnp.dot(G, h, preferred_element_type=jnp.float32),
                axis=1, keepdims=True)
    q = jnp.sum(jnp.dot(G, h * h, preferred_element_type=jnp.float32),
                axis=1, keepdims=True)
    v = q - m * m
    mr = jnp.dot(U, m, preferred_element_type=jnp.float32)
    vr = jnp.dot(U, v, preferred_element_type=jnp.float32)
    return (h - mr) * jax.lax.rsqrt(vr + eps) * gamma + beta


def _silu(x):
    return x * jax.nn.sigmoid(x)


def _moe_kernel(S, HID, C,
                x_ref, dw9_ref, g1_ref, b1_ref, pw1_ref, g2_ref, b2_ref,
                pw2_ref, pb_ref, ew1_ref, eg1_ref, eb1_ref, ew2_ref,
                eg2_ref, eb2_ref, pm_ref, tt_ref, g1m_ref, u1m_ref,
                g2m_ref, u2m_ref, ge1_ref, ue1_ref, ge2_ref, ue2_ref,
                out_ref):
    xb = x_ref[0]
    p = jnp.dot(xb, pm_ref[...], preferred_element_type=jnp.float32)
    dw9 = dw9_ref[...]
    h = jnp.zeros(p.shape, jnp.float32)
    for t in range(9):
        h = h + dw9[:, t:t + 1] * jnp.dot(
            p, tt_ref[t * S:(t + 1) * S, :],
            preferred_element_type=jnp.float32)
    h = _silu(_gn(h, g1m_ref[...], u1m_ref[...], g1_ref[...], b1_ref[...]))
    h = jnp.dot(pw1_ref[...], h, preferred_element_type=jnp.float32)
    h = _silu(_gn(h, g2m_ref[...], u2m_ref[...], g2_ref[...], b2_ref[...]))
    lm = jnp.dot(pw2_ref[...], h, preferred_element_type=jnp.float32)
    logits = jnp.sum(lm, axis=1, keepdims=True) / S + pb_ref[...]
    mx = jnp.max(logits)
    ex = jnp.exp(logits - mx)
    probs = ex / jnp.sum(ex)
    idxc = jax.lax.broadcasted_iota(jnp.int32, probs.shape, 0)
    v1 = jnp.max(probs)
    i1 = jnp.min(jnp.where(probs >= v1, idxc, 10000))
    probs2 = jnp.where(idxc == i1, -1.0, probs)
    v2 = jnp.max(probs2)
    i2 = jnp.min(jnp.where(probs2 >= v2, idxc, 10000))
    s = v1 + v2
    w1 = v1 / (s + 1e-9)
    w2 = v2 / (s + 1e-9)
    w1 = jnp.where(w1 > 0.01, w1, 0.0)
    w2 = jnp.where(w2 > 0.01, w2, 0.0)
    acc = jnp.zeros(xb.shape, jnp.float32)
    for ei, wi in ((i1, w1), (i2, w2)):
        we1 = ew1_ref[pl.ds(ei, 1), :, :].reshape(HID, C)
        hd = jnp.dot(we1, xb, preferred_element_type=jnp.float32)
        eg1 = eg1_ref[pl.ds(ei, 1), :, :].reshape(HID, 1)
        eb1 = eb1_ref[pl.ds(ei, 1), :, :].reshape(HID, 1)
        hd = _silu(_gn(hd, ge1_ref[...], ue1_ref[...], eg1, eb1))
        we2 = ew2_ref[pl.ds(ei, 1), :, :].reshape(C, HID)
        od = jnp.dot(we2, hd, preferred_element_type=jnp.float32)
        eg2 = eg2_ref[pl.ds(ei, 1), :, :].reshape(C, 1)
        eb2 = eb2_ref[pl.ds(ei, 1), :, :].reshape(C, 1)
        od = _gn(od, ge2_ref[...], ue2_ref[...], eg2, eb2)
        acc = acc + wi * od
    out_ref[0] = acc


def kernel(x, r_dw, r_gn1_g, r_gn1_b, r_pw1, r_gn2_g, r_gn2_b,
           r_pw2_w, r_pw2_b, e_w1, e_gn1_g, e_gn1_b, e_w2, e_gn2_g, e_gn2_b):
    B, C, H, W = x.shape
    E, HID = e_w1.shape[0], e_w1.shape[1]
    RED = r_pw1.shape[0]
    PS = 8
    S = (H // PS) * (W // PS)
    HW = H * W

    consts = _np_constants(C, H, W, PS, RED, HID, E)
    consts = tuple(jnp.asarray(c) for c in consts)

    x_r = x.reshape(B, C, HW)
    dw9 = r_dw.reshape(C, 9)
    ins = (x_r, dw9,
           r_gn1_g.reshape(C, 1), r_gn1_b.reshape(C, 1),
           r_pw1.reshape(RED, C),
           r_gn2_g.reshape(RED, 1), r_gn2_b.reshape(RED, 1),
           r_pw2_w.reshape(E, RED), r_pw2_b.reshape(E, 1),
           e_w1.reshape(E, HID, C),
           e_gn1_g.reshape(E, HID, 1), e_gn1_b.reshape(E, HID, 1),
           e_w2.reshape(E, C, HID),
           e_gn2_g.reshape(E, C, 1), e_gn2_b.reshape(E, C, 1)) + consts

    def full_spec(a):
        nd = a.ndim
        return pl.BlockSpec(a.shape, lambda b, _n=nd: (0,) * _n)

    in_specs = [pl.BlockSpec((1, C, HW), lambda b: (b, 0, 0))]
    in_specs += [full_spec(a) for a in ins[1:]]

    import functools
    body = functools.partial(_moe_kernel, S, HID, C)
    out = pl.pallas_call(
        body,
        grid=(B,),
        in_specs=in_specs,
        out_specs=pl.BlockSpec((1, C, HW), lambda b: (b, 0, 0)),
        out_shape=jax.ShapeDtypeStruct((B, C, HW), jnp.float32),
    )(*ins)
    return out.reshape(B, C, H, W)
